# baseline (speedup 1.0000x reference)
import functools
import jax
import jax.numpy as jnp
from jax import lax
from jax.experimental import pallas as pl
from jax.experimental.pallas import tpu as pltpu
from jax.experimental.pallas import tpu_sc as plsc

_B = 4
_S = 4096
_D = 1024
_NW = 32
_SPW = _S // _NW
_R = 16
_NJ = _SPW // _R
_NG = _NJ * _B


def _sc_body(x_hbm, pos_hbm, out_hbm,
             xb0, xb1, idxb, sin0, sin1, sadd0, sadd1, sout0, sout1):
    xbufs = (xb0, xb1)
    sins = (sin0, sin1)
    sadds = (sadd0, sadd1)
    souts = (sout0, sout1)

    wid = lax.axis_index("s") * 2 + lax.axis_index("c")
    base_seq = wid * _SPW
    iota = lax.iota(jnp.int32, 16)

    def seq_row(g):
        j = g // _B
        return base_seq + j * _R

    def x_row(g):
        j, b = divmod(g, _B)
        return b * _S + base_seq + j * _R

    in_d = [None] * _NG
    add_d = [None] * _NG
    out_d = [None] * _NG

    in_d[0] = pltpu.async_copy(
        x_hbm.at[pl.ds(x_row(0), _R), :], xbufs[0], sins[0])

    for g in range(_NG):
        buf = g % 2
        in_d[g].wait()
        if g + 1 < _NG:
            if g - 1 >= 0:
                out_d[g - 1].wait()
            in_d[g + 1] = pltpu.async_copy(
                x_hbm.at[pl.ds(x_row(g + 1), _R), :],
                xbufs[(g + 1) % 2], sins[(g + 1) % 2])
        idxb[...] = seq_row(g) + iota
        add_d[g] = pltpu.async_copy(
            pos_hbm.at[idxb], xbufs[buf], sadds[buf], add=True)
        add_d[g].wait()
        out_d[g] = pltpu.async_copy(
            xbufs[buf], out_hbm.at[pl.ds(x_row(g), _R), :], souts[buf])

    out_d[_NG - 2].wait()
    out_d[_NG - 1].wait()


def kernel(x, pos_table):
    batch, seq_len, d_model = x.shape
    xf = x.reshape(batch * seq_len, d_model)
    posf = pos_table[:seq_len]

    mesh = plsc.VectorSubcoreMesh(core_axis_name="c", subcore_axis_name="s")
    k = functools.partial(
        pl.kernel,
        mesh=mesh,
        compiler_params=pltpu.CompilerParams(use_tc_tiling_on_sc=False),
        out_type=jax.ShapeDtypeStruct((batch * seq_len, d_model), x.dtype),
        scratch_types=[
            pltpu.VMEM((_R, _D), jnp.float32),
            pltpu.VMEM((_R, _D), jnp.float32),
            pltpu.VMEM((_R,), jnp.int32),
            pltpu.SemaphoreType.DMA,
            pltpu.SemaphoreType.DMA,
            pltpu.SemaphoreType.DMA,
            pltpu.SemaphoreType.DMA,
            pltpu.SemaphoreType.DMA,
            pltpu.SemaphoreType.DMA,
        ],
    )(_sc_body)
    out = k(xf, posf)
    return out.reshape(batch, seq_len, d_model)


# speedup vs baseline: 1.0783x; 1.0783x over previous
import functools
import jax
import jax.numpy as jnp
from jax import lax
from jax.experimental import pallas as pl
from jax.experimental.pallas import tpu as pltpu
from jax.experimental.pallas import tpu_sc as plsc

_B = 4
_S = 4096
_D = 1024
_NW = 32
_SPW = _S // _NW
_R = 32
_NJ = _SPW // _R
_NG = _NJ * _B
_NBUF = 3


def _sc_body(x_hbm, pos_hbm, out_hbm,
             xb0, xb1, xb2, idxb,
             sin0, sin1, sin2, sadd0, sadd1, sadd2, sout0, sout1, sout2):
    xbufs = (xb0, xb1, xb2)
    sins = (sin0, sin1, sin2)
    sadds = (sadd0, sadd1, sadd2)
    souts = (sout0, sout1, sout2)

    wid = lax.axis_index("s") * 2 + lax.axis_index("c")
    base_seq = wid * _SPW
    iota = lax.iota(jnp.int32, 16)

    for j in range(_NJ):
        for k in range(_R // 16):
            idxb[j, pl.ds(k * 16, 16)] = base_seq + j * _R + k * 16 + iota

    def rows(g):
        j, b = divmod(g, _B)
        return b, base_seq + j * _R

    in_d = [None] * _NG
    add_d = [None] * _NG
    out_d = [None] * _NG

    def start_in(g):
        b, r = rows(g)
        in_d[g] = pltpu.async_copy(
            x_hbm.at[b, pl.ds(r, _R), :], xbufs[g % _NBUF], sins[g % _NBUF])

    start_in(0)
    if _NG > 1:
        start_in(1)

    for g in range(_NG):
        buf = g % _NBUF
        j, b = divmod(g, _B)
        in_d[g].wait()
        add_d[g] = pltpu.async_copy(
            pos_hbm.at[idxb.at[j]], xbufs[buf], sadds[buf], add=True)
        if g + 2 < _NG:
            if g - 1 >= 0:
                out_d[g - 1].wait()
            start_in(g + 2)
        add_d[g].wait()
        b2, r2 = rows(g)
        out_d[g] = pltpu.async_copy(
            xbufs[buf], out_hbm.at[b2, pl.ds(r2, _R), :], souts[buf])

    for g in range(_NG - _NBUF, _NG):
        out_d[g].wait()


def kernel(x, pos_table):
    batch, seq_len, d_model = x.shape

    mesh = plsc.VectorSubcoreMesh(core_axis_name="c", subcore_axis_name="s")
    k = functools.partial(
        pl.kernel,
        mesh=mesh,
        compiler_params=pltpu.CompilerParams(use_tc_tiling_on_sc=False),
        out_type=jax.ShapeDtypeStruct((batch, seq_len, d_model), x.dtype),
        scratch_types=[
            pltpu.VMEM((_R, _D), jnp.float32),
            pltpu.VMEM((_R, _D), jnp.float32),
            pltpu.VMEM((_R, _D), jnp.float32),
            pltpu.VMEM((_NJ, _R), jnp.int32),
            pltpu.SemaphoreType.DMA,
            pltpu.SemaphoreType.DMA,
            pltpu.SemaphoreType.DMA,
            pltpu.SemaphoreType.DMA,
            pltpu.SemaphoreType.DMA,
            pltpu.SemaphoreType.DMA,
            pltpu.SemaphoreType.DMA,
            pltpu.SemaphoreType.DMA,
            pltpu.SemaphoreType.DMA,
        ],
    )(_sc_body)
    return k(x, pos_table)


# speedup vs baseline: 3.2437x; 3.0083x over previous
import functools
import jax
import jax.numpy as jnp
from jax import lax
from jax.experimental import pallas as pl
from jax.experimental.pallas import tpu as pltpu
from jax.experimental.pallas import tpu_sc as plsc

_B = 4
_S = 4096
_D = 1024
_NW = 32
_SPW = _S // _NW
_R = 16
_NJ = _SPW // _R
_NG = _NJ * _B


def _sc_body(x_hbm, pos_hbm, out_hbm,
             xb0, xb1, xb2, ob0, ob1, pb0, pb1,
             sin0, sin1, sin2, sout0, sout1, sp0, sp1):
    xbufs = (xb0, xb1, xb2)
    obufs = (ob0, ob1)
    pbufs = (pb0, pb1)
    sins = (sin0, sin1, sin2)
    souts = (sout0, sout1)
    sps = (sp0, sp1)

    wid = lax.axis_index("s") * 2 + lax.axis_index("c")
    base_seq = wid * _SPW

    def rows(g):
        j, b = divmod(g, _B)
        return b, base_seq + j * _R

    in_d = [None] * _NG
    out_d = [None] * _NG
    p_d = [None] * _NJ

    def start_in(g):
        b, r = rows(g)
        in_d[g] = pltpu.async_copy(
            x_hbm.at[b, pl.ds(r, _R), :], xbufs[g % 3], sins[g % 3])

    def start_pos(j):
        p_d[j] = pltpu.async_copy(
            pos_hbm.at[pl.ds(base_seq + j * _R, _R), :],
            pbufs[j % 2], sps[j % 2])

    start_pos(0)
    start_in(0)
    start_in(1)

    for g in range(_NG):
        j, b = divmod(g, _B)
        in_d[g].wait()
        if b == 0:
            p_d[j].wait()
            if j + 1 < _NJ:
                start_pos(j + 1)
        if g + 2 < _NG:
            start_in(g + 2)
        if g - 2 >= 0:
            out_d[g - 2].wait()

        xb = xbufs[g % 3]
        ob = obufs[g % 2]
        pb = pbufs[j % 2]

        def row_body(row, _):
            @plsc.parallel_loop(0, _D, step=16, unroll=16)
            def _add(i):
                sl = pl.ds(i, 16)
                ob[row, sl] = xb[row, sl] + pb[row, sl]
            return 0

        lax.fori_loop(0, _R, row_body, 0)

        b2, r2 = rows(g)
        out_d[g] = pltpu.async_copy(
            ob, out_hbm.at[b2, pl.ds(r2, _R), :], souts[g % 2])

    out_d[_NG - 2].wait()
    out_d[_NG - 1].wait()


def kernel(x, pos_table):
    batch, seq_len, d_model = x.shape

    mesh = plsc.VectorSubcoreMesh(core_axis_name="c", subcore_axis_name="s")
    k = functools.partial(
        pl.kernel,
        mesh=mesh,
        out_type=jax.ShapeDtypeStruct((batch, seq_len, d_model), x.dtype),
        scratch_types=[
            pltpu.VMEM((_R, _D), jnp.float32),
            pltpu.VMEM((_R, _D), jnp.float32),
            pltpu.VMEM((_R, _D), jnp.float32),
            pltpu.VMEM((_R, _D), jnp.float32),
            pltpu.VMEM((_R, _D), jnp.float32),
            pltpu.VMEM((_R, _D), jnp.float32),
            pltpu.VMEM((_R, _D), jnp.float32),
            pltpu.SemaphoreType.DMA,
            pltpu.SemaphoreType.DMA,
            pltpu.SemaphoreType.DMA,
            pltpu.SemaphoreType.DMA,
            pltpu.SemaphoreType.DMA,
            pltpu.SemaphoreType.DMA,
            pltpu.SemaphoreType.DMA,
        ],
    )(_sc_body)
    return k(x, pos_table)
